# baseline (speedup 1.0000x reference)
import functools

import jax
import jax.numpy as jnp
from jax import lax
from jax.experimental import pallas as pl
from jax.experimental.pallas import tpu as pltpu
from jax.experimental.pallas import tpu_sc as plsc

VOCAB_SIZE = 1000000
EMB_DIM = 32
BATCH_SIZE = 16384
V4 = VOCAB_SIZE // 4


def _build_sc_gather():
    info = plsc.get_sparse_core_info()
    num_cores, num_subcores = info.num_cores, info.num_subcores
    num_workers = num_cores * num_subcores
    b_per_w = BATCH_SIZE // num_workers
    half = b_per_w // 2
    mesh = plsc.VectorSubcoreMesh(core_axis_name="c", subcore_axis_name="s")

    @functools.partial(
        pl.kernel,
        mesh=mesh,
        compiler_params=pltpu.CompilerParams(needs_layout_passes=False),
        out_type=[
            jax.ShapeDtypeStruct((BATCH_SIZE * EMB_DIM,), jnp.float32),
            jax.ShapeDtypeStruct((BATCH_SIZE * EMB_DIM,), jnp.float32),
        ],
        scratch_types=[
            pltpu.VMEM((b_per_w,), jnp.int32),
            pltpu.VMEM((b_per_w,), jnp.int32),
            pltpu.VMEM((b_per_w,), jnp.int32),
            pltpu.VMEM((b_per_w,), jnp.int32),
            pltpu.VMEM((half, 128), jnp.float32),
            pltpu.VMEM((half, 128), jnp.float32),
            pltpu.VMEM((b_per_w * EMB_DIM,), jnp.float32),
            pltpu.VMEM((b_per_w * EMB_DIM,), jnp.float32),
            pltpu.SemaphoreType.DMA,
            pltpu.SemaphoreType.DMA,
        ],
    )
    def sc_gather(targets_hbm, contexts_hbm, ttable_hbm, ctable_hbm,
                  tout_hbm, cout_hbm,
                  tidx_v, cidx_v, tgidx_v, cgidx_v, trows_v, crows_v,
                  tsel_v, csel_v, sem_t, sem_c):
        wid = lax.axis_index("s") * num_cores + lax.axis_index("c")
        base = wid * b_per_w
        pltpu.sync_copy(targets_hbm.at[pl.ds(base, b_per_w)], tidx_v)
        pltpu.sync_copy(contexts_hbm.at[pl.ds(base, b_per_w)], cidx_v)

        def shift(g, carry):
            s = pl.ds(g * 16, 16)
            tgidx_v[s] = tidx_v[s] >> 2
            cgidx_v[s] = cidx_v[s] >> 2
            return carry
        lax.fori_loop(0, b_per_w // 16, shift, 0)

        lanes = lax.iota(jnp.int32, 16)

        def select(idx_ref, rows_ref, sel_ref, h):
            def body(g, carry):
                r16 = g * 16 + lanes
                idx16 = idx_ref[pl.ds(h * half + g * 16, 16)]
                src_col0 = (idx16 & 3) * 32
                dst0 = (h * half) * EMB_DIM + r16 * EMB_DIM
                for j in range(EMB_DIM):
                    vals = plsc.load_gather(rows_ref, [r16, src_col0 + j])
                    plsc.store_scatter(sel_ref, [dst0 + j], vals)
                return carry
            lax.fori_loop(0, half // 16, body, 0)

        for h in range(2):
            cp_t = pltpu.async_copy(
                ttable_hbm.at[tgidx_v.at[pl.ds(h * half, half)]],
                trows_v, sem_t)
            cp_c = pltpu.async_copy(
                ctable_hbm.at[cgidx_v.at[pl.ds(h * half, half)]],
                crows_v, sem_c)
            cp_t.wait()
            select(tidx_v, trows_v, tsel_v, h)
            cp_c.wait()
            select(cidx_v, crows_v, csel_v, h)

        out_sz = b_per_w * EMB_DIM
        pltpu.sync_copy(tsel_v, tout_hbm.at[pl.ds(wid * out_sz, out_sz)])
        pltpu.sync_copy(csel_v, cout_hbm.at[pl.ds(wid * out_sz, out_sz)])

    return sc_gather


_sc_gather = _build_sc_gather()


@jax.jit
def kernel(targets, contexts, target_table, context_table):
    t128 = target_table.reshape(V4, 128)
    c128 = context_table.reshape(V4, 128)
    t_emb, c_emb = _sc_gather(
        targets.astype(jnp.int32), contexts.astype(jnp.int32), t128, c128)
    return (t_emb.reshape(BATCH_SIZE, EMB_DIM),
            c_emb.reshape(BATCH_SIZE, EMB_DIM))
